# baseline (speedup 1.0000x reference)
import functools

import jax
import jax.numpy as jnp
from jax import lax
from jax.experimental import pallas as pl
from jax.experimental.pallas import tpu as pltpu
from jax.experimental.pallas import tpu_sc as plsc

B = 16384
D = 10
L = 16
NS = 16
ROWS_PER_SC = B // NS
ELEMS_PER_SC = ROWS_PER_SC * D
NVREG = ELEMS_PER_SC // L
UNROLL = 4


def _body(o_hbm, t_hbm, l_hbm, out_hbm,
          lbl_v, validf_v, o_v, t_v, part_v, loc_v, res_v, shared):
    sid = lax.axis_index("s")
    row0 = sid * ROWS_PER_SC
    e0 = sid * ELEMS_PER_SC

    pltpu.sync_copy(o_hbm.at[pl.ds(e0, ELEMS_PER_SC)], o_v)
    pltpu.sync_copy(t_hbm.at[pl.ds(e0, ELEMS_PER_SC)], t_v)
    pltpu.sync_copy(l_hbm.at[pl.ds(row0, ROWS_PER_SC)], lbl_v)

    ones = jnp.full((L,), 1.0, jnp.float32)
    zeros = jnp.full((L,), 0.0, jnp.float32)
    c_acc = zeros
    for i in range(ROWS_PER_SC // L):
        lbl = lbl_v[pl.ds(i * L, L)]
        v = jnp.where(lbl == -2, ones, zeros)
        validf_v[pl.ds(i * L, L)] = v
        c_acc = c_acc + v

    iota = lax.iota(jnp.int32, L)
    tens = jnp.full((L,), D, jnp.int32)

    def step(j, accs):
        new = []
        for u in range(UNROLL):
            base = (j * UNROLL + u) * L
            o = o_v[pl.ds(base, L)]
            t = t_v[pl.ds(base, L)]
            d = o - t
            rows = lax.div(iota + base, tens)
            m = plsc.load_gather(validf_v, [rows])
            new.append(accs[u] + d * d * m)
        return tuple(new)

    accs = lax.fori_loop(0, NVREG // UNROLL, step,
                         (zeros, zeros, zeros, zeros))
    s_acc = accs[0] + accs[1] + accs[2] + accs[3]

    part_v[pl.ds(0, L)] = s_acc
    part_v[pl.ds(L, L)] = c_acc
    pltpu.sync_copy(part_v, shared.at[pl.ds(sid * 2 * L, 2 * L)])
    plsc.subcore_barrier()

    @pl.when(sid == 0)
    def _():
        pltpu.sync_copy(shared, loc_v)
        s_tot = zeros
        c_tot = zeros
        for i in range(NS):
            s_tot = s_tot + loc_v[pl.ds(i * 2 * L, L)]
            c_tot = c_tot + loc_v[pl.ds(i * 2 * L + L, L)]
        ts = jnp.sum(s_tot)
        tc = jnp.sum(c_tot)
        res_v[...] = jnp.full((L,), ts, jnp.float32) / jnp.full(
            (L,), tc, jnp.float32)
        pltpu.sync_copy(res_v, out_hbm)


_sc_call = functools.partial(
    pl.kernel,
    mesh=plsc.VectorSubcoreMesh(core_axis_name="c", subcore_axis_name="s",
                                num_cores=1),
    out_type=jax.ShapeDtypeStruct((L,), jnp.float32),
    compiler_params=pltpu.CompilerParams(needs_layout_passes=False),
    scratch_types=[
        pltpu.VMEM((ROWS_PER_SC,), jnp.int32),
        pltpu.VMEM((ROWS_PER_SC,), jnp.float32),
        pltpu.VMEM((ELEMS_PER_SC,), jnp.float32),
        pltpu.VMEM((ELEMS_PER_SC,), jnp.float32),
        pltpu.VMEM((2 * L,), jnp.float32),
        pltpu.VMEM((NS * 2 * L,), jnp.float32),
        pltpu.VMEM((L,), jnp.float32),
        pltpu.VMEM_SHARED((NS * 2 * L,), jnp.float32),
    ],
)(_body)


@jax.jit
def kernel(landmark_out, landmark_target, label):
    o_flat = landmark_out.reshape(-1)
    t_flat = landmark_target.reshape(-1)
    l_flat = label.reshape(-1)
    out = _sc_call(o_flat, t_flat, l_flat)
    return out[0]


# speedup vs baseline: 2.2454x; 2.2454x over previous
import functools

import jax
import jax.numpy as jnp
from jax import lax
from jax.experimental import pallas as pl
from jax.experimental.pallas import tpu as pltpu
from jax.experimental.pallas import tpu_sc as plsc

B = 16384
D = 10
L = 16
NS = 16
ROWS_PER_SC = B // NS
NGROUP = ROWS_PER_SC // L


def _body(o_hbm, t_hbm, l_hbm, out_hbm,
          lbl_v, o_v, t_v, part_v, loc_v, res_v, shared, sem):
    sid = lax.axis_index("s")
    row0 = sid * ROWS_PER_SC

    cps = []
    for c in range(D):
        cps.append(pltpu.async_copy(
            o_hbm.at[pl.ds(c * B + row0, ROWS_PER_SC)],
            o_v.at[pl.ds(c * ROWS_PER_SC, ROWS_PER_SC)], sem))
        cps.append(pltpu.async_copy(
            t_hbm.at[pl.ds(c * B + row0, ROWS_PER_SC)],
            t_v.at[pl.ds(c * ROWS_PER_SC, ROWS_PER_SC)], sem))
    cps.append(pltpu.async_copy(
        l_hbm.at[pl.ds(row0, ROWS_PER_SC)], lbl_v, sem))
    for cp in cps:
        cp.wait()

    ones = jnp.full((L,), 1.0, jnp.float32)
    zeros = jnp.full((L,), 0.0, jnp.float32)

    def step(g, carry):
        s_acc, c_acc = carry
        base = g * L
        lbl = lbl_v[pl.ds(base, L)]
        vf = jnp.where(lbl == -2, ones, zeros)
        sq = zeros
        for c in range(D):
            o = o_v[pl.ds(c * ROWS_PER_SC + base, L)]
            t = t_v[pl.ds(c * ROWS_PER_SC + base, L)]
            d = o - t
            sq = sq + d * d
        return (s_acc + sq * vf, c_acc + vf)

    s_acc, c_acc = lax.fori_loop(0, NGROUP, step, (zeros, zeros))

    part_v[pl.ds(0, L)] = s_acc
    part_v[pl.ds(L, L)] = c_acc
    pltpu.sync_copy(part_v, shared.at[pl.ds(sid * 2 * L, 2 * L)])
    plsc.subcore_barrier()

    @pl.when(sid == 0)
    def _():
        pltpu.sync_copy(shared, loc_v)
        s_tot = zeros
        c_tot = zeros
        for i in range(NS):
            s_tot = s_tot + loc_v[pl.ds(i * 2 * L, L)]
            c_tot = c_tot + loc_v[pl.ds(i * 2 * L + L, L)]
        ts = jnp.sum(s_tot)
        tc = jnp.sum(c_tot)
        res_v[...] = jnp.full((L,), ts, jnp.float32) / jnp.full(
            (L,), tc, jnp.float32)
        pltpu.sync_copy(res_v, out_hbm)


_sc_call = functools.partial(
    pl.kernel,
    mesh=plsc.VectorSubcoreMesh(core_axis_name="c", subcore_axis_name="s",
                                num_cores=1),
    out_type=jax.ShapeDtypeStruct((L,), jnp.float32),
    compiler_params=pltpu.CompilerParams(needs_layout_passes=False),
    scratch_types=[
        pltpu.VMEM((ROWS_PER_SC,), jnp.int32),
        pltpu.VMEM((ROWS_PER_SC * D,), jnp.float32),
        pltpu.VMEM((ROWS_PER_SC * D,), jnp.float32),
        pltpu.VMEM((2 * L,), jnp.float32),
        pltpu.VMEM((NS * 2 * L,), jnp.float32),
        pltpu.VMEM((L,), jnp.float32),
        pltpu.VMEM_SHARED((NS * 2 * L,), jnp.float32),
        pltpu.SemaphoreType.DMA,
    ],
)(_body)


@jax.jit
def kernel(landmark_out, landmark_target, label):
    o_flat = landmark_out.T.reshape(-1)
    t_flat = landmark_target.T.reshape(-1)
    l_flat = label.reshape(-1)
    out = _sc_call(o_flat, t_flat, l_flat)
    return out[0]


# speedup vs baseline: 2.2692x; 1.0106x over previous
import functools

import jax
import jax.numpy as jnp
from jax import lax
from jax.experimental import pallas as pl
from jax.experimental.pallas import tpu as pltpu
from jax.experimental.pallas import tpu_sc as plsc

B = 16384
D = 10
L = 16
NS = 16
ROWS_PER_SC = B // NS
NGROUP = ROWS_PER_SC // L


def _body(x_hbm, l_hbm, out_hbm,
          lbl_v, o_v, t_v, part_v, loc_v, res_v, shared, sem):
    sid = lax.axis_index("s")
    row0 = sid * ROWS_PER_SC

    cps = []
    for c in range(D):
        cps.append(pltpu.async_copy(
            x_hbm.at[pl.ds(c * B + row0, ROWS_PER_SC)],
            o_v.at[pl.ds(c * ROWS_PER_SC, ROWS_PER_SC)], sem))
        cps.append(pltpu.async_copy(
            x_hbm.at[pl.ds(D * B + c * B + row0, ROWS_PER_SC)],
            t_v.at[pl.ds(c * ROWS_PER_SC, ROWS_PER_SC)], sem))
    cps.append(pltpu.async_copy(
        l_hbm.at[pl.ds(row0, ROWS_PER_SC)], lbl_v, sem))
    for cp in cps:
        cp.wait()

    ones = jnp.full((L,), 1.0, jnp.float32)
    zeros = jnp.full((L,), 0.0, jnp.float32)

    def step(g, carry):
        s_acc, c_acc = carry
        base = g * L
        lbl = lbl_v[pl.ds(base, L)]
        vf = jnp.where(lbl == -2, ones, zeros)
        sq = zeros
        for c in range(D):
            o = o_v[pl.ds(c * ROWS_PER_SC + base, L)]
            t = t_v[pl.ds(c * ROWS_PER_SC + base, L)]
            d = o - t
            sq = sq + d * d
        return (s_acc + sq * vf, c_acc + vf)

    s_acc, c_acc = lax.fori_loop(0, NGROUP, step, (zeros, zeros))

    part_v[pl.ds(0, L)] = s_acc
    part_v[pl.ds(L, L)] = c_acc
    pltpu.sync_copy(part_v, shared.at[pl.ds(sid * 2 * L, 2 * L)])
    plsc.subcore_barrier()

    @pl.when(sid == 0)
    def _():
        pltpu.sync_copy(shared, loc_v)
        s_tot = zeros
        c_tot = zeros
        for i in range(NS):
            s_tot = s_tot + loc_v[pl.ds(i * 2 * L, L)]
            c_tot = c_tot + loc_v[pl.ds(i * 2 * L + L, L)]
        ts = jnp.sum(s_tot)
        tc = jnp.sum(c_tot)
        res_v[...] = jnp.full((L,), ts, jnp.float32) / jnp.full(
            (L,), tc, jnp.float32)
        pltpu.sync_copy(res_v, out_hbm)


_sc_call = functools.partial(
    pl.kernel,
    mesh=plsc.VectorSubcoreMesh(core_axis_name="c", subcore_axis_name="s",
                                num_cores=1),
    out_type=jax.ShapeDtypeStruct((L,), jnp.float32),
    compiler_params=pltpu.CompilerParams(needs_layout_passes=False),
    scratch_types=[
        pltpu.VMEM((ROWS_PER_SC,), jnp.int32),
        pltpu.VMEM((ROWS_PER_SC * D,), jnp.float32),
        pltpu.VMEM((ROWS_PER_SC * D,), jnp.float32),
        pltpu.VMEM((2 * L,), jnp.float32),
        pltpu.VMEM((NS * 2 * L,), jnp.float32),
        pltpu.VMEM((L,), jnp.float32),
        pltpu.VMEM_SHARED((NS * 2 * L,), jnp.float32),
        pltpu.SemaphoreType.DMA,
    ],
)(_body)


@jax.jit
def kernel(landmark_out, landmark_target, label):
    x_flat = jnp.stack([landmark_out.T, landmark_target.T]).reshape(-1)
    l_flat = label.reshape(-1)
    out = _sc_call(x_flat, l_flat)
    return out[0]
